# Initial kernel scaffold; baseline (speedup 1.0000x reference)
import jax
import jax.numpy as jnp
from jax.experimental import pallas as pl


def kernel(x, edge_index, edge_attr, batch, W1, b1, W2, b2, Wlin, blin):
    raise NotImplementedError("write your pallas kernel here")


# speedup vs baseline: 12.3202x; 12.3202x over previous
import functools

import jax
import jax.numpy as jnp
from jax import lax
from jax.experimental import pallas as pl
from jax.experimental.pallas import tpu as pltpu
from jax.experimental.pallas import tpu_sc as plsc

N_NODES = 10000
N_EDGES = 320000
D_IN = 128
D_HID = 128
D_OUT = 10
N_GRAPHS = 64

NC = 2
NS = 16
CHUNK = 128
N_CHUNKS = N_EDGES // CHUNK
PER_SC = N_CHUNKS // NC
ROW_BLK = 1000
N_BLKS = N_NODES // ROW_BLK

_sc_mesh = plsc.VectorSubcoreMesh(
    core_axis_name="c", subcore_axis_name="s", num_cores=NC, num_subcores=NS)


@functools.partial(
    pl.kernel,
    out_type=jax.ShapeDtypeStruct((NC, N_NODES), jnp.float32),
    mesh=_sc_mesh,
    scratch_types=[
        pltpu.VMEM_SHARED((N_NODES,), jnp.float32),
        pltpu.VMEM((CHUNK,), jnp.int32),
        pltpu.VMEM((CHUNK,), jnp.float32),
    ],
)
def _sc_deg(col_hbm, ew_hbm, zeros_hbm, degp_hbm, deg_sh, idx_v, val_v):
    c = lax.axis_index("c")
    s = lax.axis_index("s")
    wid = c * NS + s

    @pl.when(s == 0)
    def _():
        pltpu.sync_copy(zeros_hbm, deg_sh)

    plsc.subcore_barrier()

    n_my = jnp.where(wid < N_CHUNKS % (NC * NS),
                     N_CHUNKS // (NC * NS) + 1, N_CHUNKS // (NC * NS))

    def body(k, _):
        base = (wid + k * NC * NS) * CHUNK
        pltpu.sync_copy(col_hbm.at[pl.ds(base, CHUNK)], idx_v)
        pltpu.sync_copy(ew_hbm.at[pl.ds(base, CHUNK)], val_v)
        pltpu.sync_copy(val_v, deg_sh.at[idx_v], add=True)
        return 0

    lax.fori_loop(0, n_my, body, 0)
    plsc.subcore_barrier()

    @pl.when(s == 0)
    def _():
        pltpu.sync_copy(deg_sh, degp_hbm.at[c])


@functools.partial(
    pl.kernel,
    out_type=jax.ShapeDtypeStruct((NC, N_NODES, D_HID), jnp.float32),
    mesh=_sc_mesh,
    scratch_types=[
        pltpu.VMEM_SHARED((N_NODES, D_HID), jnp.float32),
        pltpu.VMEM((CHUNK,), jnp.int32),
        pltpu.VMEM((CHUNK,), jnp.int32),
        pltpu.VMEM((CHUNK,), jnp.float32),
        pltpu.VMEM((CHUNK, D_HID), jnp.float32),
        pltpu.SemaphoreType.DMA,
    ],
)
def _sc_agg(y_hbm, row_hbm, col_hbm, ew_hbm, zeros_hbm, aggp_hbm,
            accum_sh, idxr_v, idxc_v, ew_v, rows_v, sem):
    c = lax.axis_index("c")
    s = lax.axis_index("s")

    @pl.when(jnp.logical_and(s == 0, c == 0))
    def _():
        pltpu.sync_copy(y_hbm, accum_sh)

    @pl.when(jnp.logical_and(s == 0, c == 1))
    def _():
        pltpu.sync_copy(zeros_hbm, accum_sh)

    plsc.subcore_barrier()

    n_my = jnp.where(s < PER_SC % NS, PER_SC // NS + 1, PER_SC // NS)

    def body(k, _):
        base = (c * PER_SC + s + k * NS) * CHUNK
        pltpu.sync_copy(row_hbm.at[pl.ds(base, CHUNK)], idxr_v)
        pltpu.sync_copy(col_hbm.at[pl.ds(base, CHUNK)], idxc_v)
        pltpu.sync_copy(ew_hbm.at[pl.ds(base, CHUNK)], ew_v)
        pltpu.async_copy(y_hbm.at[idxr_v], rows_v, sem).wait()

        def scale(g, _):
            wvec = ew_v[pl.ds(g * 16, 16)]
            for i in range(16):
                w = wvec[i]
                r = g * 16 + i
                for j in range(D_HID // 16):
                    sl = pl.ds(j * 16, 16)
                    rows_v[r, sl] = rows_v[r, sl] * w
            return 0

        lax.fori_loop(0, CHUNK // 16, scale, 0)
        pltpu.sync_copy(rows_v, accum_sh.at[idxc_v], add=True)
        return 0

    lax.fori_loop(0, n_my, body, 0)
    plsc.subcore_barrier()

    @pl.when(s == 0)
    def _():
        pltpu.sync_copy(accum_sh, aggp_hbm.at[c])


def _dinv_of(degp_ref):
    deg = degp_ref[0, 0, :] + degp_ref[0, 1, :] + 1.0
    return lax.rsqrt(deg)


def _tc_y1_body(x_ref, w_ref, degp_ref, y_ref):
    dinv = _dinv_of(degp_ref)
    y_ref[...] = jnp.dot(x_ref[...], w_ref[...],
                         preferred_element_type=jnp.float32) * dinv[:, None]


def _tc_y1(x, W1, degp):
    return pl.pallas_call(
        _tc_y1_body,
        grid=(N_BLKS,),
        in_specs=[
            pl.BlockSpec((ROW_BLK, D_IN), lambda i: (i, 0)),
            pl.BlockSpec((D_IN, D_HID), lambda i: (0, 0)),
            pl.BlockSpec((1, NC, ROW_BLK), lambda i: (i, 0, 0)),
        ],
        out_specs=pl.BlockSpec((ROW_BLK, D_HID), lambda i: (i, 0)),
        out_shape=jax.ShapeDtypeStruct((N_NODES, D_HID), jnp.float32),
    )(x, W1, degp)


def _tc_y2_body(aggp_ref, degp_ref, b_ref, w_ref, y_ref):
    dinv = _dinv_of(degp_ref)
    agg = aggp_ref[0, :, :] + aggp_ref[1, :, :]
    h = jnp.maximum(agg * dinv[:, None] + b_ref[...], 0.0)
    y_ref[...] = jnp.dot(h, w_ref[...],
                         preferred_element_type=jnp.float32) * dinv[:, None]


def _tc_y2(aggp, degp, b1, W2):
    return pl.pallas_call(
        _tc_y2_body,
        grid=(N_BLKS,),
        in_specs=[
            pl.BlockSpec((NC, ROW_BLK, D_HID), lambda i: (0, i, 0)),
            pl.BlockSpec((1, NC, ROW_BLK), lambda i: (i, 0, 0)),
            pl.BlockSpec((1, D_HID), lambda i: (0, 0)),
            pl.BlockSpec((D_HID, D_HID), lambda i: (0, 0)),
        ],
        out_specs=pl.BlockSpec((ROW_BLK, D_HID), lambda i: (i, 0)),
        out_shape=jax.ShapeDtypeStruct((N_NODES, D_HID), jnp.float32),
    )(aggp, degp, b1, W2)


def _tc_final_body(aggp_ref, degp_ref, b_ref, batch_ref, wl_ref, bl_ref,
                   out_ref, psum, pcnt):
    i = pl.program_id(0)

    @pl.when(i == 0)
    def _():
        psum[...] = jnp.zeros_like(psum)
        pcnt[...] = jnp.zeros_like(pcnt)

    dinv = _dinv_of(degp_ref)
    agg = aggp_ref[0, :, :] + aggp_ref[1, :, :]
    h = jnp.maximum(agg * dinv[:, None] + b_ref[...], 0.0)
    seg = batch_ref[0, :, :]
    gids = lax.broadcasted_iota(jnp.int32, (N_GRAPHS, ROW_BLK), 0)
    onehot = jnp.where(gids == seg, 1.0, 0.0)
    psum[...] += jnp.dot(onehot, h, preferred_element_type=jnp.float32)
    pcnt[...] += jnp.sum(onehot, axis=1, keepdims=True)

    @pl.when(i == N_BLKS - 1)
    def _():
        pooled = psum[...] / jnp.maximum(pcnt[...], 1.0)
        out_ref[...] = jnp.dot(pooled, wl_ref[...],
                               preferred_element_type=jnp.float32) + bl_ref[...]


def _tc_final(aggp, degp, b2, batch3, Wlin, blin):
    return pl.pallas_call(
        _tc_final_body,
        grid=(N_BLKS,),
        in_specs=[
            pl.BlockSpec((NC, ROW_BLK, D_HID), lambda i: (0, i, 0)),
            pl.BlockSpec((1, NC, ROW_BLK), lambda i: (i, 0, 0)),
            pl.BlockSpec((1, D_HID), lambda i: (0, 0)),
            pl.BlockSpec((1, 1, ROW_BLK), lambda i: (i, 0, 0)),
            pl.BlockSpec((D_HID, D_OUT), lambda i: (0, 0)),
            pl.BlockSpec((1, D_OUT), lambda i: (0, 0)),
        ],
        out_specs=pl.BlockSpec((N_GRAPHS, D_OUT), lambda i: (0, 0)),
        out_shape=jax.ShapeDtypeStruct((N_GRAPHS, D_OUT), jnp.float32),
        scratch_shapes=[
            pltpu.VMEM((N_GRAPHS, D_HID), jnp.float32),
            pltpu.VMEM((N_GRAPHS, 1), jnp.float32),
        ],
    )(aggp, degp, b2, batch3, Wlin, blin)


@jax.jit
def kernel(x, edge_index, edge_attr, batch, W1, b1, W2, b2, Wlin, blin):
    row = edge_index[0].astype(jnp.int32)
    col = edge_index[1].astype(jnp.int32)
    ew = edge_attr.astype(jnp.float32)
    batch3 = batch.astype(jnp.int32).reshape(N_BLKS, 1, ROW_BLK)
    zeros1 = jnp.zeros((N_NODES,), jnp.float32)
    zeros2 = jnp.zeros((N_NODES, D_HID), jnp.float32)

    degp = _sc_deg(col, ew, zeros1)
    degp3 = degp.reshape(NC, N_BLKS, ROW_BLK).transpose(1, 0, 2)
    y1 = _tc_y1(x, W1, degp3)
    aggp1 = _sc_agg(y1, row, col, ew, zeros2)
    y2 = _tc_y2(aggp1, degp3, b1.reshape(1, D_HID), W2)
    aggp2 = _sc_agg(y2, row, col, ew, zeros2)
    return _tc_final(aggp2, degp3, b2.reshape(1, D_HID), batch3,
                     Wlin, blin.reshape(1, D_OUT))


# speedup vs baseline: 20.6173x; 1.6735x over previous
import functools

import jax
import jax.numpy as jnp
from jax import lax
from jax.experimental import pallas as pl
from jax.experimental.pallas import tpu as pltpu
from jax.experimental.pallas import tpu_sc as plsc

N_NODES = 10000
N_EDGES = 320000
D_IN = 128
D_HID = 128
D_OUT = 10
N_GRAPHS = 64

NC = 2
NS = 16
CHUNK = 128
CH_PAD = 2560
E_PAD = CH_PAD * CHUNK
PER_SC = CH_PAD // NC
T_AGG = PER_SC // NS
T_DEG = CH_PAD // (NC * NS)
NBUF = 2
NBUF_D = 8
ROW_BLK = 1000
N_BLKS = N_NODES // ROW_BLK

_sc_mesh = plsc.VectorSubcoreMesh(
    core_axis_name="c", subcore_axis_name="s", num_cores=NC, num_subcores=NS)


@functools.partial(
    pl.kernel,
    out_type=jax.ShapeDtypeStruct((NC, N_NODES), jnp.float32),
    mesh=_sc_mesh,
    scratch_types=[
        pltpu.VMEM_SHARED((N_NODES,), jnp.float32),
        pltpu.VMEM((NBUF_D, 2, CHUNK), jnp.int32),
        pltpu.VMEM((NBUF_D, CHUNK), jnp.float32),
        pltpu.SemaphoreType.DMA((NBUF_D,)),
        pltpu.SemaphoreType.DMA((NBUF_D,)),
        pltpu.SemaphoreType.DMA((NBUF_D,)),
    ],
)
def _sc_deg(pki_hbm, pkw_hbm, zeros_hbm, degp_hbm,
            deg_sh, pki_v, pkw_v, sem_i, sem_w, sem_s):
    c = lax.axis_index("c")
    s = lax.axis_index("s")
    wid = c * NS + s

    @pl.when(s == 0)
    def _():
        pltpu.sync_copy(zeros_hbm, deg_sh)

    plsc.subcore_barrier()

    def outer(t, _):
        loads = []
        for b in range(NBUF_D):
            cid = wid + (t * NBUF_D + b) * NC * NS
            di = pltpu.async_copy(pki_hbm.at[cid], pki_v.at[b], sem_i.at[b])
            dw = pltpu.async_copy(pkw_hbm.at[cid], pkw_v.at[b], sem_w.at[b])
            loads.append((di, dw))
        scats = []
        for b in range(NBUF_D):
            loads[b][0].wait()
            loads[b][1].wait()
            ss = pltpu.async_copy(
                pkw_v.at[b], deg_sh.at[pki_v.at[b].at[1]], sem_s.at[b],
                add=True)
            scats.append(ss)
        for ss in scats:
            ss.wait()
        return 0

    lax.fori_loop(0, T_DEG // NBUF_D, outer, 0)
    plsc.subcore_barrier()

    @pl.when(s == 0)
    def _():
        pltpu.sync_copy(deg_sh, degp_hbm.at[c])


@functools.partial(
    pl.kernel,
    out_type=jax.ShapeDtypeStruct((NC, N_NODES, D_HID), jnp.float32),
    mesh=_sc_mesh,
    scratch_types=[
        pltpu.VMEM_SHARED((N_NODES, D_HID), jnp.float32),
        pltpu.VMEM((NBUF, 2, CHUNK), jnp.int32),
        pltpu.VMEM((NBUF, CHUNK), jnp.float32),
        pltpu.VMEM((NBUF, CHUNK, D_HID), jnp.float32),
        pltpu.SemaphoreType.DMA((NBUF,)),
        pltpu.SemaphoreType.DMA((NBUF,)),
        pltpu.SemaphoreType.DMA((NBUF,)),
        pltpu.SemaphoreType.DMA((NBUF,)),
    ],
)
def _sc_agg(y_hbm, pki_hbm, pkw_hbm, zeros_hbm, aggp_hbm,
            accum_sh, pki_v, pkw_v, rows_v, sem_i, sem_w, sem_g, sem_s):
    c = lax.axis_index("c")
    s = lax.axis_index("s")

    @pl.when(jnp.logical_and(s == 0, c == 0))
    def _():
        pltpu.sync_copy(y_hbm, accum_sh)

    @pl.when(jnp.logical_and(s == 0, c == 1))
    def _():
        pltpu.sync_copy(zeros_hbm, accum_sh)

    plsc.subcore_barrier()

    def outer(t, _):
        loads = []
        for b in range(NBUF):
            cid = c * PER_SC + s + (t * NBUF + b) * NS
            di = pltpu.async_copy(pki_hbm.at[cid], pki_v.at[b], sem_i.at[b])
            dw = pltpu.async_copy(pkw_hbm.at[cid], pkw_v.at[b], sem_w.at[b])
            loads.append((di, dw))
        gathers = []
        for b in range(NBUF):
            loads[b][0].wait()
            g = pltpu.async_copy(
                y_hbm.at[pki_v.at[b].at[0]], rows_v.at[b], sem_g.at[b])
            gathers.append(g)
        scats = []
        for b in range(NBUF):
            gathers[b].wait()
            loads[b][1].wait()

            def scale(g2, _, b=b):
                wvec = pkw_v[b, pl.ds(g2 * 16, 16)]
                for i in range(16):
                    w = wvec[i]
                    r = g2 * 16 + i
                    for j in range(D_HID // 16):
                        sl = pl.ds(j * 16, 16)
                        rows_v[b, r, sl] = rows_v[b, r, sl] * w
                return 0

            lax.fori_loop(0, CHUNK // 16, scale, 0)
            ss = pltpu.async_copy(
                rows_v.at[b], accum_sh.at[pki_v.at[b].at[1]], sem_s.at[b],
                add=True)
            scats.append(ss)
        for ss in scats:
            ss.wait()
        return 0

    lax.fori_loop(0, T_AGG // NBUF, outer, 0)
    plsc.subcore_barrier()

    @pl.when(s == 0)
    def _():
        pltpu.sync_copy(accum_sh, aggp_hbm.at[c])


def _dinv_of(degp_ref):
    deg = degp_ref[0, 0, :] + degp_ref[0, 1, :] + 1.0
    return lax.rsqrt(deg)


def _tc_y1_body(x_ref, w_ref, degp_ref, y_ref):
    dinv = _dinv_of(degp_ref)
    y_ref[...] = jnp.dot(x_ref[...], w_ref[...],
                         preferred_element_type=jnp.float32) * dinv[:, None]


def _tc_y1(x, W1, degp):
    return pl.pallas_call(
        _tc_y1_body,
        grid=(N_BLKS,),
        in_specs=[
            pl.BlockSpec((ROW_BLK, D_IN), lambda i: (i, 0)),
            pl.BlockSpec((D_IN, D_HID), lambda i: (0, 0)),
            pl.BlockSpec((1, NC, ROW_BLK), lambda i: (i, 0, 0)),
        ],
        out_specs=pl.BlockSpec((ROW_BLK, D_HID), lambda i: (i, 0)),
        out_shape=jax.ShapeDtypeStruct((N_NODES, D_HID), jnp.float32),
    )(x, W1, degp)


def _tc_y2_body(aggp_ref, degp_ref, b_ref, w_ref, y_ref):
    dinv = _dinv_of(degp_ref)
    agg = aggp_ref[0, :, :] + aggp_ref[1, :, :]
    h = jnp.maximum(agg * dinv[:, None] + b_ref[...], 0.0)
    y_ref[...] = jnp.dot(h, w_ref[...],
                         preferred_element_type=jnp.float32) * dinv[:, None]


def _tc_y2(aggp, degp, b1, W2):
    return pl.pallas_call(
        _tc_y2_body,
        grid=(N_BLKS,),
        in_specs=[
            pl.BlockSpec((NC, ROW_BLK, D_HID), lambda i: (0, i, 0)),
            pl.BlockSpec((1, NC, ROW_BLK), lambda i: (i, 0, 0)),
            pl.BlockSpec((1, D_HID), lambda i: (0, 0)),
            pl.BlockSpec((D_HID, D_HID), lambda i: (0, 0)),
        ],
        out_specs=pl.BlockSpec((ROW_BLK, D_HID), lambda i: (i, 0)),
        out_shape=jax.ShapeDtypeStruct((N_NODES, D_HID), jnp.float32),
    )(aggp, degp, b1, W2)


def _tc_final_body(aggp_ref, degp_ref, b_ref, batch_ref, wl_ref, bl_ref,
                   out_ref, psum, pcnt):
    i = pl.program_id(0)

    @pl.when(i == 0)
    def _():
        psum[...] = jnp.zeros_like(psum)
        pcnt[...] = jnp.zeros_like(pcnt)

    dinv = _dinv_of(degp_ref)
    agg = aggp_ref[0, :, :] + aggp_ref[1, :, :]
    h = jnp.maximum(agg * dinv[:, None] + b_ref[...], 0.0)
    seg = batch_ref[0, :, :]
    gids = lax.broadcasted_iota(jnp.int32, (N_GRAPHS, ROW_BLK), 0)
    onehot = jnp.where(gids == seg, 1.0, 0.0)
    psum[...] += jnp.dot(onehot, h, preferred_element_type=jnp.float32)
    pcnt[...] += jnp.sum(onehot, axis=1, keepdims=True)

    @pl.when(i == N_BLKS - 1)
    def _():
        pooled = psum[...] / jnp.maximum(pcnt[...], 1.0)
        out_ref[...] = jnp.dot(pooled, wl_ref[...],
                               preferred_element_type=jnp.float32) + bl_ref[...]


def _tc_final(aggp, degp, b2, batch3, Wlin, blin):
    return pl.pallas_call(
        _tc_final_body,
        grid=(N_BLKS,),
        in_specs=[
            pl.BlockSpec((NC, ROW_BLK, D_HID), lambda i: (0, i, 0)),
            pl.BlockSpec((1, NC, ROW_BLK), lambda i: (i, 0, 0)),
            pl.BlockSpec((1, D_HID), lambda i: (0, 0)),
            pl.BlockSpec((1, 1, ROW_BLK), lambda i: (i, 0, 0)),
            pl.BlockSpec((D_HID, D_OUT), lambda i: (0, 0)),
            pl.BlockSpec((1, D_OUT), lambda i: (0, 0)),
        ],
        out_specs=pl.BlockSpec((N_GRAPHS, D_OUT), lambda i: (0, 0)),
        out_shape=jax.ShapeDtypeStruct((N_GRAPHS, D_OUT), jnp.float32),
        scratch_shapes=[
            pltpu.VMEM((N_GRAPHS, D_HID), jnp.float32),
            pltpu.VMEM((N_GRAPHS, 1), jnp.float32),
        ],
    )(aggp, degp, b2, batch3, Wlin, blin)


@jax.jit
def kernel(x, edge_index, edge_attr, batch, W1, b1, W2, b2, Wlin, blin):
    row = edge_index[0].astype(jnp.int32)
    col = edge_index[1].astype(jnp.int32)
    ew = edge_attr.astype(jnp.float32)
    batch3 = batch.astype(jnp.int32).reshape(N_BLKS, 1, ROW_BLK)
    zeros1 = jnp.zeros((N_NODES,), jnp.float32)
    zeros2 = jnp.zeros((N_NODES, D_HID), jnp.float32)

    pad = E_PAD - N_EDGES
    padidx = jnp.arange(pad, dtype=jnp.int32) % N_NODES
    rowp = jnp.concatenate([row, padidx]).reshape(CH_PAD, CHUNK)
    colp = jnp.concatenate([col, padidx]).reshape(CH_PAD, CHUNK)
    ewp = jnp.concatenate([ew, jnp.zeros((pad,), jnp.float32)])
    pki = jnp.stack([rowp, colp], axis=1)
    pkw = ewp.reshape(CH_PAD, CHUNK)

    degp = _sc_deg(pki, pkw, zeros1)
    degp3 = degp.reshape(NC, N_BLKS, ROW_BLK).transpose(1, 0, 2)
    y1 = _tc_y1(x, W1, degp3)
    aggp1 = _sc_agg(y1, pki, pkw, zeros2)
    y2 = _tc_y2(aggp1, degp3, b1.reshape(1, D_HID), W2)
    aggp2 = _sc_agg(y2, pki, pkw, zeros2)
    return _tc_final(aggp2, degp3, b2.reshape(1, D_HID), batch3,
                     Wlin, blin.reshape(1, D_OUT))
